# baseline (speedup 1.0000x reference)
import math

import jax
import jax.numpy as jnp
from jax.experimental import pallas as pl
from jax.experimental.pallas import tpu as pltpu

_H = 16
_FACTOR = 5

_HI = jax.lax.Precision.HIGHEST
_MED = jax.lax.Precision.HIGH


def _qk_kernel(x_ref, wqt_ref, a_ref, qk_ref):
    q = jnp.dot(x_ref[0], wqt_ref[...],
                preferred_element_type=jnp.float32)
    qk_ref[0] = jnp.dot(q, a_ref[0],
                        preferred_element_type=jnp.float32)


def _scores_kernel(p_ref, xth_ref, xtl_ref, a_ref, *, scale):
    pb = p_ref[0]
    ph = pb.astype(jnp.bfloat16)
    plo = (pb - ph.astype(jnp.float32)).astype(jnp.bfloat16)
    xth = xth_ref[0]
    xtl = xtl_ref[0]
    sc = jnp.dot(ph, xtl, preferred_element_type=jnp.float32)
    sc += jnp.dot(plo, xth, preferred_element_type=jnp.float32)
    sc += jnp.dot(ph, xth, preferred_element_type=jnp.float32)
    sc = sc * scale
    mx = jnp.max(sc, axis=1, keepdims=True)
    e = jnp.exp(sc - mx)
    a_ref[0] = e / jnp.sum(e, axis=1, keepdims=True)


def _apply_kernel(a_ref, x_ref, t_ref):
    t_ref[0] = jnp.dot(a_ref[0], x_ref[0],
                       preferred_element_type=jnp.float32)


def _topk_kernel(m_ref, idx_ref, *, u, length):
    m = m_ref[...]
    iota = jax.lax.broadcasted_iota(jnp.int32, m.shape, 1)
    cols = []
    for _ in range(u):
        mx = jnp.max(m, axis=1, keepdims=True)
        am = jnp.min(jnp.where(m == mx, iota, length), axis=1)
        cols.append(am[:, None])
        m = jnp.where(iota == am[:, None], -jnp.inf, m)
    idx_ref[...] = jnp.concatenate(cols, axis=1)


def _assemble_kernel(idx_ref, base_ref, delta_ref, out_ref, *, hu, length):
    b = pl.program_id(0)
    out_ref[0] = jnp.broadcast_to(base_ref[0], (length, base_ref.shape[-1]))

    def body(j, carry):
        r = idx_ref[b * hu + j]
        out_ref[0, pl.ds(r, 1), :] = (out_ref[0, pl.ds(r, 1), :]
                                      + delta_ref[0, pl.ds(j, 1), :])
        return carry

    jax.lax.fori_loop(0, hu, body, 0)


def kernel(x, W_q, W_k, W_v, W_out, idx_sample):
    B, L, dm = x.shape
    H = _H
    D = dm // H
    scale = 1.0 / math.sqrt(D)
    u = min(L, max(1, int(_FACTOR * math.log(L + 1))))
    S = idx_sample.shape[0]
    HU = H * u

    Wq_r = W_q.reshape(H, D, dm)
    Wk_r = W_k.reshape(H, D, dm)
    Wv_r = W_v.reshape(H, D, dm)
    Wo_r = W_out.reshape(dm, H, D)

    xs = x[:, idx_sample, :]
    Ks = xs @ W_k.T
    KsT = Ks.reshape(B, S, H, D).transpose(0, 2, 3, 1)
    A_bd = (KsT[:, :, :, None, :]
            * jnp.eye(H, dtype=jnp.float32)[None, :, None, :, None]
            ).reshape(B, dm, H * S)

    TL = 256
    WqT = W_q.T
    qk_raw = pl.pallas_call(
        _qk_kernel,
        grid=(B, L // TL),
        in_specs=[
            pl.BlockSpec((1, TL, dm), lambda b, t: (b, t, 0)),
            pl.BlockSpec((dm, dm), lambda b, t: (0, 0)),
            pl.BlockSpec((1, dm, H * S), lambda b, t: (b, 0, 0)),
        ],
        out_specs=pl.BlockSpec((1, TL, H * S), lambda b, t: (b, t, 0)),
        out_shape=jax.ShapeDtypeStruct((B, L, H * S), jnp.float32),
    )(x, WqT, A_bd).reshape(B, L, H, S)

    QK_sample = qk_raw * scale
    M = (jnp.max(QK_sample, axis=-1)
         - jnp.mean(QK_sample, axis=-1)).transpose(0, 2, 1)
    RB = 16
    idx = pl.pallas_call(
        lambda mr, ir: _topk_kernel(mr, ir, u=u, length=L),
        grid=(B * H // RB,),
        in_specs=[pl.BlockSpec((RB, L), lambda t: (t, 0))],
        out_specs=pl.BlockSpec((RB, u), lambda t: (t, 0)),
        out_shape=jax.ShapeDtypeStruct((B * H, u), jnp.int32),
    )(M.reshape(B * H, L)).reshape(B, H, u)

    xg = x[jnp.arange(B)[:, None, None], idx]
    Qr = jnp.einsum('bhum,hdm->bhud', xg, Wq_r, precision=_MED)
    P = jnp.einsum('bhud,hdm->bhum', Qr, Wk_r,
                   precision=_MED).reshape(B, HU, dm)

    TQ = 304 if HU % 304 == 0 else HU
    xTh = jnp.swapaxes(x, 1, 2).astype(jnp.bfloat16)
    xTl = jnp.swapaxes(
        x - x.astype(jnp.bfloat16).astype(jnp.float32), 1, 2
    ).astype(jnp.bfloat16)
    attn = pl.pallas_call(
        lambda pr, xhr, xlr, ar: _scores_kernel(pr, xhr, xlr, ar, scale=scale),
        grid=(B, HU // TQ),
        in_specs=[
            pl.BlockSpec((1, TQ, dm), lambda b, t: (b, t, 0)),
            pl.BlockSpec((1, dm, L), lambda b, t: (b, 0, 0)),
            pl.BlockSpec((1, dm, L), lambda b, t: (b, 0, 0)),
        ],
        out_specs=pl.BlockSpec((1, TQ, L), lambda b, t: (b, t, 0)),
        out_shape=jax.ShapeDtypeStruct((B, HU, L), jnp.float32),
    )(P, xTh, xTl)
    T = pl.pallas_call(
        _apply_kernel,
        grid=(B, HU // TQ),
        in_specs=[
            pl.BlockSpec((1, TQ, L), lambda b, t: (b, t, 0)),
            pl.BlockSpec((1, L, dm), lambda b, t: (b, 0, 0)),
        ],
        out_specs=pl.BlockSpec((1, TQ, dm), lambda b, t: (b, t, 0)),
        out_shape=jax.ShapeDtypeStruct((B, HU, dm), jnp.float32),
    )(attn, x)

    V_top = jnp.einsum('bhum,hdm->bhud', T.reshape(B, H, u, dm), Wv_r,
                       precision=_MED)
    mean_x = jnp.mean(x, axis=1)
    meanV = jnp.einsum('bm,nm->bn', mean_x, W_v, precision=_MED)
    base = jnp.einsum('bm,nm->bn', meanV, W_out, precision=_MED)
    dv = V_top - meanV.reshape(B, 1, H, D).transpose(0, 2, 1, 3)
    delta = jnp.einsum('bhud,mhd->bhum', dv, Wo_r, precision=_MED)

    out = pl.pallas_call(
        lambda ir, br, dr, orf: _assemble_kernel(ir, br, dr, orf,
                                                 hu=HU, length=L),
        grid_spec=pltpu.PrefetchScalarGridSpec(
            num_scalar_prefetch=1,
            grid=(B,),
            in_specs=[
                pl.BlockSpec((1, 1, dm), lambda b, ir: (b, 0, 0)),
                pl.BlockSpec((1, HU, dm), lambda b, ir: (b, 0, 0)),
            ],
            out_specs=pl.BlockSpec((1, L, dm), lambda b, ir: (b, 0, 0)),
        ),
        out_shape=jax.ShapeDtypeStruct((B, L, dm), jnp.float32),
    )(idx.reshape(B * HU), base.reshape(B, 1, dm), delta.reshape(B, HU, dm))
    return out


# speedup vs baseline: 1.1116x; 1.1116x over previous
import math

import jax
import jax.numpy as jnp
from jax.experimental import pallas as pl
from jax.experimental.pallas import tpu as pltpu

_H = 16
_FACTOR = 5

_HI = jax.lax.Precision.HIGHEST
_MED = jax.lax.Precision.HIGH


def _qk_kernel(x_ref, wqt_ref, kst_ref, qk_ref, *, n_heads, hd):
    q = jnp.dot(x_ref[0], wqt_ref[...],
                preferred_element_type=jnp.float32)
    for h in range(n_heads):
        qk_ref[0, :, h, :] = jnp.dot(q[:, h * hd:(h + 1) * hd], kst_ref[0, h],
                                     preferred_element_type=jnp.float32)


def _scores_kernel(p_ref, xth_ref, xtl_ref, a_ref, *, scale):
    pb = p_ref[0]
    ph = pb.astype(jnp.bfloat16)
    plo = (pb - ph.astype(jnp.float32)).astype(jnp.bfloat16)
    xth = xth_ref[0]
    xtl = xtl_ref[0]
    sc = jnp.dot(ph, xtl, preferred_element_type=jnp.float32)
    sc += jnp.dot(plo, xth, preferred_element_type=jnp.float32)
    sc += jnp.dot(ph, xth, preferred_element_type=jnp.float32)
    sc = sc * scale
    mx = jnp.max(sc, axis=1, keepdims=True)
    e = jnp.exp(sc - mx)
    a_ref[0] = e / jnp.sum(e, axis=1, keepdims=True)


def _apply_kernel(a_ref, x_ref, t_ref):
    t_ref[0] = jnp.dot(a_ref[0], x_ref[0],
                       preferred_element_type=jnp.float32)


def _topk_kernel(m_ref, idx_ref, *, u, length):
    m = m_ref[...]
    iota = jax.lax.broadcasted_iota(jnp.int32, m.shape, 1)
    cols = []
    for _ in range(u):
        mx = jnp.max(m, axis=1, keepdims=True)
        am = jnp.min(jnp.where(m == mx, iota, length), axis=1)
        cols.append(am[:, None])
        m = jnp.where(iota == am[:, None], -jnp.inf, m)
    idx_ref[...] = jnp.concatenate(cols, axis=1)


def _assemble_kernel(idx_ref, base_ref, delta_ref, out_ref, *, hu, length):
    b = pl.program_id(0)
    out_ref[0] = jnp.broadcast_to(base_ref[0], (length, base_ref.shape[-1]))

    def body(j, carry):
        r = idx_ref[b * hu + j]
        out_ref[0, pl.ds(r, 1), :] = (out_ref[0, pl.ds(r, 1), :]
                                      + delta_ref[0, pl.ds(j, 1), :])
        return carry

    jax.lax.fori_loop(0, hu, body, 0)


def kernel(x, W_q, W_k, W_v, W_out, idx_sample):
    B, L, dm = x.shape
    H = _H
    D = dm // H
    scale = 1.0 / math.sqrt(D)
    u = min(L, max(1, int(_FACTOR * math.log(L + 1))))
    S = idx_sample.shape[0]
    HU = H * u

    Wq_r = W_q.reshape(H, D, dm)
    Wk_r = W_k.reshape(H, D, dm)
    Wv_r = W_v.reshape(H, D, dm)
    Wo_r = W_out.reshape(dm, H, D)

    xs = x[:, idx_sample, :]
    Ks = xs @ W_k.T
    KsT = Ks.reshape(B, S, H, D).transpose(0, 2, 3, 1)

    TL = 256
    WqT = W_q.T
    qk_raw = pl.pallas_call(
        lambda xr, wr, kr, qr: _qk_kernel(xr, wr, kr, qr, n_heads=H, hd=D),
        grid=(B, L // TL),
        in_specs=[
            pl.BlockSpec((1, TL, dm), lambda b, t: (b, t, 0)),
            pl.BlockSpec((dm, dm), lambda b, t: (0, 0)),
            pl.BlockSpec((1, H, D, S), lambda b, t: (b, 0, 0, 0)),
        ],
        out_specs=pl.BlockSpec((1, TL, H, S), lambda b, t: (b, t, 0, 0)),
        out_shape=jax.ShapeDtypeStruct((B, L, H, S), jnp.float32),
    )(x, WqT, KsT)

    QK_sample = qk_raw * scale
    M = (jnp.max(QK_sample, axis=-1)
         - jnp.mean(QK_sample, axis=-1)).transpose(0, 2, 1)
    RB = 16
    idx = pl.pallas_call(
        lambda mr, ir: _topk_kernel(mr, ir, u=u, length=L),
        grid=(B * H // RB,),
        in_specs=[pl.BlockSpec((RB, L), lambda t: (t, 0))],
        out_specs=pl.BlockSpec((RB, u), lambda t: (t, 0)),
        out_shape=jax.ShapeDtypeStruct((B * H, u), jnp.int32),
    )(M.reshape(B * H, L)).reshape(B, H, u)

    xg = x[jnp.arange(B)[:, None, None], idx]
    Qr = jnp.einsum('bhum,hdm->bhud', xg, Wq_r, precision=_MED)
    P = jnp.einsum('bhud,hdm->bhum', Qr, Wk_r,
                   precision=_MED).reshape(B, HU, dm)

    TQ = 304 if HU % 304 == 0 else HU
    xTh = jnp.swapaxes(x, 1, 2).astype(jnp.bfloat16)
    xTl = jnp.swapaxes(
        x - x.astype(jnp.bfloat16).astype(jnp.float32), 1, 2
    ).astype(jnp.bfloat16)
    attn = pl.pallas_call(
        lambda pr, xhr, xlr, ar: _scores_kernel(pr, xhr, xlr, ar, scale=scale),
        grid=(B, HU // TQ),
        in_specs=[
            pl.BlockSpec((1, TQ, dm), lambda b, t: (b, t, 0)),
            pl.BlockSpec((1, dm, L), lambda b, t: (b, 0, 0)),
            pl.BlockSpec((1, dm, L), lambda b, t: (b, 0, 0)),
        ],
        out_specs=pl.BlockSpec((1, TQ, L), lambda b, t: (b, t, 0)),
        out_shape=jax.ShapeDtypeStruct((B, HU, L), jnp.float32),
    )(P, xTh, xTl)
    T = pl.pallas_call(
        _apply_kernel,
        grid=(B, HU // TQ),
        in_specs=[
            pl.BlockSpec((1, TQ, L), lambda b, t: (b, t, 0)),
            pl.BlockSpec((1, L, dm), lambda b, t: (b, 0, 0)),
        ],
        out_specs=pl.BlockSpec((1, TQ, dm), lambda b, t: (b, t, 0)),
        out_shape=jax.ShapeDtypeStruct((B, HU, dm), jnp.float32),
    )(attn, x)

    V_top = jnp.einsum('bhum,hdm->bhud', T.reshape(B, H, u, dm), Wv_r,
                       precision=_MED)
    mean_x = jnp.mean(x, axis=1)
    meanV = jnp.einsum('bm,nm->bn', mean_x, W_v, precision=_MED)
    base = jnp.einsum('bm,nm->bn', meanV, W_out, precision=_MED)
    dv = V_top - meanV.reshape(B, 1, H, D).transpose(0, 2, 1, 3)
    delta = jnp.einsum('bhud,mhd->bhum', dv, Wo_r, precision=_MED)

    out = pl.pallas_call(
        lambda ir, br, dr, orf: _assemble_kernel(ir, br, dr, orf,
                                                 hu=HU, length=L),
        grid_spec=pltpu.PrefetchScalarGridSpec(
            num_scalar_prefetch=1,
            grid=(B,),
            in_specs=[
                pl.BlockSpec((1, 1, dm), lambda b, ir: (b, 0, 0)),
                pl.BlockSpec((1, HU, dm), lambda b, ir: (b, 0, 0)),
            ],
            out_specs=pl.BlockSpec((1, L, dm), lambda b, ir: (b, 0, 0)),
        ),
        out_shape=jax.ShapeDtypeStruct((B, L, dm), jnp.float32),
    )(idx.reshape(B * HU), base.reshape(B, 1, dm), delta.reshape(B, HU, dm))
    return out
